# baseline (speedup 1.0000x reference)
import functools

import jax
import jax.numpy as jnp
from jax import lax
from jax.experimental import pallas as pl
from jax.experimental.pallas import tpu as pltpu
from jax.experimental.pallas import tpu_sc as plsc

N_ROWS = 1024
N_COLS = 100000

_R_SC = 512
_R_TC = N_ROWS - _R_SC

_BR = 16
_NBLK_TC = _R_TC // _BR


def _tc_body(x_ref, t_ref, out_ref):
    i = pl.program_id(0)
    x = x_ref[...]
    t = t_ref[...]
    col = lax.broadcasted_iota(jnp.int32, (_BR, N_COLS), 1)
    v = jnp.max(jnp.where(col == t, x, -jnp.inf), axis=1, keepdims=True)
    contrib = (x > v) | ((x == v) & (col < t))
    rank = jnp.sum(contrib.astype(jnp.float32), axis=1, keepdims=True)
    top1 = jnp.sum((rank < 0.5).astype(jnp.float32))
    top5 = jnp.sum((rank < 4.5).astype(jnp.float32))
    part = jnp.concatenate([top1.reshape(1, 1), top5.reshape(1, 1)], axis=1)

    @pl.when(i == 0)
    def _():
        out_ref[...] = part

    @pl.when(i > 0)
    def _():
        out_ref[...] += part


def _tc_count(pred, t2):
    return pl.pallas_call(
        _tc_body,
        grid=(_NBLK_TC,),
        in_specs=[
            pl.BlockSpec((_BR, N_COLS), lambda i: (i, 0)),
            pl.BlockSpec((_BR, 1), lambda i: (i, 0)),
        ],
        out_specs=pl.BlockSpec((1, 2), lambda i: (0, 0)),
        out_shape=jax.ShapeDtypeStruct((1, 2), jnp.float32),
    )(pred, t2)


_NC = 2
_NS = 16
_L = 16
_NW = _NC * _NS
_RPW = _R_SC // _NW
_WIN = 6272
_NWIN = N_COLS // _WIN
_TAILW = 6016


def _sc_window_count(buf, wbase, width, v_spl, t_spl, r8, acc, bound=None):
    iota = lax.iota(jnp.int32, _L)

    def it(k, a):
        b = k * _L
        x = buf[r8, pl.ds(b, _L)]
        cols = (wbase + b) + iota
        gt = x > v_spl
        if bound is not None:
            gt = gt & (cols < bound)
        tie = (x == v_spl) & (cols < t_spl)
        return a + jnp.where(gt | tie, 1, 0)

    def it4(k, a):
        for u in range(4):
            a = it(k * 4 + u, a)
        return a

    n = width // _L
    n4 = n // 4
    acc = lax.fori_loop(0, n4, it4, acc)
    for k in range(n4 * 4, n):
        acc = it(k, acc)
    return acc


def _sc_body(pred_hbm, tgt_hbm, out_hbm, t_v, chunk_v, buf_v, val_v, sem):
    wid = lax.axis_index("s") * _NC + lax.axis_index("c")
    base = _R_SC * 0 + _R_TC + wid * _RPW
    pltpu.sync_copy(tgt_hbm.at[pl.ds(base, _RPW)], t_v)
    zero = jnp.zeros((), jnp.int32)
    t1c = zero
    t5c = zero
    iota = lax.iota(jnp.int32, _L)
    for g in range(_RPW // 8):
        rows = base + g * 8
        v_spls = []
        t_spls = []
        for r8 in range(8):
            ridx = jnp.full((_L,), g * 8 + r8, jnp.int32)
            t_spl = plsc.load_gather(t_v, [ridx])
            t_sca = jnp.max(t_spl)
            c0_sca = pl.multiple_of(
                lax.shift_left(lax.shift_right_logical(t_sca, 7), 7), 128)
            pltpu.async_copy(
                pred_hbm.at[pl.ds(rows, 8), pl.ds(c0_sca, 128)],
                chunk_v, sem).wait()
            off = t_spl - lax.shift_left(
                lax.shift_right_logical(t_spl, 7), 7)
            v_spl = plsc.load_gather(
                chunk_v, [jnp.full((_L,), r8, jnp.int32), off])
            v_spls.append(v_spl)
            t_spls.append(t_spl)

        accs = tuple(jnp.zeros((_L,), jnp.int32) for _ in range(8))

        def win(w, accs):
            wbase = pl.multiple_of(w * _WIN, 128)
            pltpu.async_copy(
                pred_hbm.at[pl.ds(rows, 8), pl.ds(wbase, _WIN)],
                buf_v, sem).wait()
            return tuple(
                _sc_window_count(buf_v, wbase, _WIN, v_spls[r8], t_spls[r8],
                                 r8, accs[r8])
                for r8 in range(8))

        accs = lax.fori_loop(0, _NWIN, win, accs)

        wbase = pl.multiple_of(
            lax.max(jnp.int32(_NWIN * _WIN), wid), 128)
        pltpu.async_copy(
            pred_hbm.at[pl.ds(rows, 8), pl.ds(wbase, _TAILW)],
            buf_v.at[:, pl.ds(0, _TAILW)], sem).wait()
        for r8 in range(8):
            acc = _sc_window_count(buf_v, wbase, _TAILW, v_spls[r8],
                                   t_spls[r8], r8, accs[r8],
                                   bound=jnp.int32(N_COLS))
            rank = jnp.sum(acc)
            t1c = t1c + jnp.where(rank < 1, 1, 0)
            t5c = t5c + jnp.where(rank < 5, 1, 0)

    val = (jnp.where(iota == 0, t1c.astype(jnp.float32), 0.0)
           + jnp.where(iota == 1, t5c.astype(jnp.float32), 0.0))
    val_v[...] = val
    pltpu.sync_copy(val_v, out_hbm.at[pl.ds(wid * _L, _L)])


def _sc_count(pred, t1d):
    mesh = plsc.VectorSubcoreMesh(core_axis_name="c", subcore_axis_name="s")
    k = functools.partial(
        pl.kernel,
        mesh=mesh,
        compiler_params=pltpu.CompilerParams(needs_layout_passes=False),
        out_type=jax.ShapeDtypeStruct((_NW * _L,), jnp.float32),
        scratch_types=[
            pltpu.VMEM((_RPW,), jnp.int32),
            pltpu.VMEM((8, 128), jnp.float32),
            pltpu.VMEM((8, _WIN), jnp.float32),
            pltpu.VMEM((_L,), jnp.float32),
            pltpu.SemaphoreType.DMA,
        ],
    )(_sc_body)
    return k(pred, t1d)


@jax.jit
def kernel(pred, target):
    t1d = target.astype(jnp.int32)
    t2 = t1d.reshape(N_ROWS, 1)
    tc_counts = _tc_count(pred, t2).reshape(2)
    sc_out = _sc_count(pred, t1d).reshape(_NW, _L)
    sc_counts = jnp.stack(
        [jnp.sum(sc_out[:, 0]), jnp.sum(sc_out[:, 1])])
    return (tc_counts + sc_counts) * (100.0 / N_ROWS)


# speedup vs baseline: 1.1957x; 1.1957x over previous
import functools

import jax
import jax.numpy as jnp
from jax import lax
from jax.experimental import pallas as pl
from jax.experimental.pallas import tpu as pltpu
from jax.experimental.pallas import tpu_sc as plsc

N_ROWS = 1024
N_COLS = 100000

_R_SC = 256
_R_TC = N_ROWS - _R_SC

_BR = 16
_NBLK_TC = _R_TC // _BR


def _tc_body(x_ref, t_ref, out_ref):
    i = pl.program_id(0)
    x = x_ref[...]
    t = t_ref[...]
    col = lax.broadcasted_iota(jnp.int32, (_BR, N_COLS), 1)
    v = jnp.max(jnp.where(col == t, x, -jnp.inf), axis=1, keepdims=True)
    contrib = (x > v) | ((x == v) & (col < t))
    rank = jnp.sum(contrib.astype(jnp.float32), axis=1, keepdims=True)
    top1 = jnp.sum((rank < 0.5).astype(jnp.float32))
    top5 = jnp.sum((rank < 4.5).astype(jnp.float32))
    part = jnp.concatenate([top1.reshape(1, 1), top5.reshape(1, 1)], axis=1)

    @pl.when(i == 0)
    def _():
        out_ref[...] = part

    @pl.when(i > 0)
    def _():
        out_ref[...] += part


def _tc_count(pred, t2):
    return pl.pallas_call(
        _tc_body,
        grid=(_NBLK_TC,),
        in_specs=[
            pl.BlockSpec((_BR, N_COLS), lambda i: (i, 0)),
            pl.BlockSpec((_BR, 1), lambda i: (i, 0)),
        ],
        out_specs=pl.BlockSpec((1, 2), lambda i: (0, 0)),
        out_shape=jax.ShapeDtypeStruct((1, 2), jnp.float32),
    )(pred, t2)


_NC = 2
_NS = 16
_L = 16
_NW = _NC * _NS
_RPW = _R_SC // _NW
_WIN = 6272
_NWIN = N_COLS // _WIN
_TAILW = 6016


def _sc_window_count(buf, wbase, width, v_spl, t_spl, r8, acc, bound=None):
    iota = lax.iota(jnp.int32, _L)

    def it(k, a):
        b = k * _L
        x = buf[r8, pl.ds(b, _L)]
        cols = (wbase + b) + iota
        gt = x > v_spl
        if bound is not None:
            gt = gt & (cols < bound)
        tie = (x == v_spl) & (cols < t_spl)
        return a + jnp.where(gt | tie, 1, 0)

    def it4(k, a):
        for u in range(4):
            a = it(k * 4 + u, a)
        return a

    n = width // _L
    n4 = n // 4
    acc = lax.fori_loop(0, n4, it4, acc)
    for k in range(n4 * 4, n):
        acc = it(k, acc)
    return acc


def _sc_body(pred_hbm, tgt_hbm, out_hbm, t_v, chunk_v, buf_v, val_v, sem):
    wid = lax.axis_index("s") * _NC + lax.axis_index("c")
    base = _R_SC * 0 + _R_TC + wid * _RPW
    pltpu.sync_copy(tgt_hbm.at[pl.ds(base, _RPW)], t_v)
    zero = jnp.zeros((), jnp.int32)
    t1c = zero
    t5c = zero
    iota = lax.iota(jnp.int32, _L)
    for g in range(_RPW // 8):
        rows = base + g * 8
        v_spls = []
        t_spls = []
        for r8 in range(8):
            ridx = jnp.full((_L,), g * 8 + r8, jnp.int32)
            t_spl = plsc.load_gather(t_v, [ridx])
            t_sca = jnp.max(t_spl)
            c0_sca = pl.multiple_of(
                lax.shift_left(lax.shift_right_logical(t_sca, 7), 7), 128)
            pltpu.async_copy(
                pred_hbm.at[pl.ds(rows, 8), pl.ds(c0_sca, 128)],
                chunk_v, sem).wait()
            off = t_spl - lax.shift_left(
                lax.shift_right_logical(t_spl, 7), 7)
            v_spl = plsc.load_gather(
                chunk_v, [jnp.full((_L,), r8, jnp.int32), off])
            v_spls.append(v_spl)
            t_spls.append(t_spl)

        accs = tuple(jnp.zeros((_L,), jnp.int32) for _ in range(8))

        def win(w, accs):
            wbase = pl.multiple_of(w * _WIN, 128)
            pltpu.async_copy(
                pred_hbm.at[pl.ds(rows, 8), pl.ds(wbase, _WIN)],
                buf_v, sem).wait()
            return tuple(
                _sc_window_count(buf_v, wbase, _WIN, v_spls[r8], t_spls[r8],
                                 r8, accs[r8])
                for r8 in range(8))

        accs = lax.fori_loop(0, _NWIN, win, accs)

        wbase = pl.multiple_of(
            lax.max(jnp.int32(_NWIN * _WIN), wid), 128)
        pltpu.async_copy(
            pred_hbm.at[pl.ds(rows, 8), pl.ds(wbase, _TAILW)],
            buf_v.at[:, pl.ds(0, _TAILW)], sem).wait()
        for r8 in range(8):
            acc = _sc_window_count(buf_v, wbase, _TAILW, v_spls[r8],
                                   t_spls[r8], r8, accs[r8],
                                   bound=jnp.int32(N_COLS))
            rank = jnp.sum(acc)
            t1c = t1c + jnp.where(rank < 1, 1, 0)
            t5c = t5c + jnp.where(rank < 5, 1, 0)

    val = (jnp.where(iota == 0, t1c.astype(jnp.float32), 0.0)
           + jnp.where(iota == 1, t5c.astype(jnp.float32), 0.0))
    val_v[...] = val
    pltpu.sync_copy(val_v, out_hbm.at[pl.ds(wid * _L, _L)])


def _sc_count(pred, t1d):
    mesh = plsc.VectorSubcoreMesh(core_axis_name="c", subcore_axis_name="s")
    k = functools.partial(
        pl.kernel,
        mesh=mesh,
        compiler_params=pltpu.CompilerParams(needs_layout_passes=False),
        out_type=jax.ShapeDtypeStruct((_NW * _L,), jnp.float32),
        scratch_types=[
            pltpu.VMEM((_RPW,), jnp.int32),
            pltpu.VMEM((8, 128), jnp.float32),
            pltpu.VMEM((8, _WIN), jnp.float32),
            pltpu.VMEM((_L,), jnp.float32),
            pltpu.SemaphoreType.DMA,
        ],
    )(_sc_body)
    return k(pred, t1d)


@jax.jit
def kernel(pred, target):
    t1d = target.astype(jnp.int32)
    t2 = t1d.reshape(N_ROWS, 1)
    tc_counts = _tc_count(pred, t2).reshape(2)
    sc_out = _sc_count(pred, t1d).reshape(_NW, _L)
    sc_counts = jnp.stack(
        [jnp.sum(sc_out[:, 0]), jnp.sum(sc_out[:, 1])])
    return (tc_counts + sc_counts) * (100.0 / N_ROWS)


# speedup vs baseline: 1.1994x; 1.0031x over previous
import functools

import jax
import jax.numpy as jnp
from jax import lax
from jax.experimental import pallas as pl
from jax.experimental.pallas import tpu as pltpu
from jax.experimental.pallas import tpu_sc as plsc

N_ROWS = 1024
N_COLS = 100000

_R_SC = 256
_R_TC = N_ROWS - _R_SC

_BR = 16
_NBLK_TC = _R_TC // _BR


def _tc_body(x_ref, t_ref, out_ref):
    i = pl.program_id(0)
    x = x_ref[...]
    t = t_ref[...]
    col = lax.broadcasted_iota(jnp.int32, (_BR, N_COLS), 1)
    v = jnp.max(jnp.where(col == t, x, -jnp.inf), axis=1, keepdims=True)
    contrib = (x > v) | ((x == v) & (col < t))
    rank = jnp.sum(contrib.astype(jnp.float32), axis=1, keepdims=True)
    top1 = jnp.sum((rank < 0.5).astype(jnp.float32))
    top5 = jnp.sum((rank < 4.5).astype(jnp.float32))
    part = jnp.concatenate([top1.reshape(1, 1), top5.reshape(1, 1)], axis=1)

    @pl.when(i == 0)
    def _():
        out_ref[...] = part

    @pl.when(i > 0)
    def _():
        out_ref[...] += part


def _tc_count(pred, t2):
    return pl.pallas_call(
        _tc_body,
        grid=(_NBLK_TC,),
        in_specs=[
            pl.BlockSpec((_BR, N_COLS), lambda i: (i, 0)),
            pl.BlockSpec((_BR, 1), lambda i: (i, 0)),
        ],
        out_specs=pl.BlockSpec((1, 2), lambda i: (0, 0)),
        out_shape=jax.ShapeDtypeStruct((1, 2), jnp.float32),
    )(pred, t2)


_NC = 2
_NS = 16
_L = 16
_NW = _NC * _NS
_RPW = _R_SC // _NW
_WIN = 6272
_NWIN = N_COLS // _WIN
_TAILW = 6016


def _sc_window_count(buf, wbase, width, v_spl, t_spl, r8, acc, bound=None):
    iota = lax.iota(jnp.int32, _L)

    def it(k, a):
        b = k * _L
        x = buf[r8, pl.ds(b, _L)]
        cols = (wbase + b) + iota
        gt = x > v_spl
        if bound is not None:
            gt = gt & (cols < bound)
        tie = (x == v_spl) & (cols < t_spl)
        return a + jnp.where(gt | tie, 1, 0)

    def it4(k, a):
        for u in range(4):
            a = it(k * 4 + u, a)
        return a

    n = width // _L
    n4 = n // 4
    acc = lax.fori_loop(0, n4, it4, acc)
    for k in range(n4 * 4, n):
        acc = it(k, acc)
    return acc


def _sc_body(pred_hbm, tgt_hbm, out_hbm, t_v, chunk_v, buf_v, val_v, sem):
    wid = lax.axis_index("s") * _NC + lax.axis_index("c")
    base = _R_TC + wid * _RPW
    pltpu.sync_copy(tgt_hbm.at[pl.ds(base, _RPW)], t_v)
    zero = jnp.zeros((), jnp.int32)
    t1c = zero
    t5c = zero
    iota = lax.iota(jnp.int32, _L)
    for g in range(_RPW // 8):
        rows = base + g * 8
        v_spls = []
        t_spls = []
        for r8 in range(8):
            ridx = jnp.full((_L,), g * 8 + r8, jnp.int32)
            t_spl = plsc.load_gather(t_v, [ridx])
            t_sca = jnp.max(t_spl)
            c0_sca = pl.multiple_of(
                lax.shift_left(lax.shift_right_logical(t_sca, 7), 7), 128)
            pltpu.async_copy(
                pred_hbm.at[pl.ds(rows, 8), pl.ds(c0_sca, 128)],
                chunk_v, sem).wait()
            off = t_spl - lax.shift_left(
                lax.shift_right_logical(t_spl, 7), 7)
            v_spl = plsc.load_gather(
                chunk_v, [jnp.full((_L,), r8, jnp.int32), off])
            v_spls.append(v_spl)
            t_spls.append(t_spl)

        accs = tuple(jnp.zeros((_L,), jnp.int32) for _ in range(8))

        def win(w, accs):
            wbase = pl.multiple_of(w * _WIN, 128)
            pltpu.async_copy(
                pred_hbm.at[pl.ds(rows, 8), pl.ds(wbase, _WIN)],
                buf_v, sem).wait()
            return tuple(
                _sc_window_count(buf_v, wbase, _WIN, v_spls[r8], t_spls[r8],
                                 r8, accs[r8])
                for r8 in range(8))

        accs = lax.fori_loop(0, _NWIN, win, accs)

        wbase = pl.multiple_of(
            lax.max(jnp.int32(_NWIN * _WIN), wid), 128)
        pltpu.async_copy(
            pred_hbm.at[pl.ds(rows, 8), pl.ds(wbase, _TAILW)],
            buf_v.at[:, pl.ds(0, _TAILW)], sem).wait()
        for r8 in range(8):
            acc = _sc_window_count(buf_v, wbase, _TAILW, v_spls[r8],
                                   t_spls[r8], r8, accs[r8],
                                   bound=jnp.int32(N_COLS))
            rank = jnp.sum(acc)
            t1c = t1c + jnp.where(rank < 1, 1, 0)
            t5c = t5c + jnp.where(rank < 5, 1, 0)

    val = (jnp.where(iota == 0, t1c.astype(jnp.float32), 0.0)
           + jnp.where(iota == 1, t5c.astype(jnp.float32), 0.0))
    val_v[...] = val
    pltpu.sync_copy(val_v, out_hbm.at[pl.ds(wid * _L, _L)])


def _sc_count(pred, t1d):
    mesh = plsc.VectorSubcoreMesh(core_axis_name="c", subcore_axis_name="s")
    k = functools.partial(
        pl.kernel,
        mesh=mesh,
        compiler_params=pltpu.CompilerParams(needs_layout_passes=False),
        out_type=jax.ShapeDtypeStruct((_NW * _L,), jnp.float32),
        scratch_types=[
            pltpu.VMEM((_RPW,), jnp.int32),
            pltpu.VMEM((8, 128), jnp.float32),
            pltpu.VMEM((8, _WIN), jnp.float32),
            pltpu.VMEM((_L,), jnp.float32),
            pltpu.SemaphoreType.DMA,
        ],
    )(_sc_body)
    return k(pred, t1d)


@jax.jit
def kernel(pred, target):
    t1d = target.astype(jnp.int32)
    t2 = t1d.reshape(N_ROWS, 1)
    tc_counts = _tc_count(pred, t2).reshape(2)
    sc_out = _sc_count(pred, t1d).reshape(_NW, _L)
    sc_counts = jnp.stack(
        [jnp.sum(sc_out[:, 0]), jnp.sum(sc_out[:, 1])])
    return (tc_counts + sc_counts) * (100.0 / N_ROWS)
